# baseline (speedup 1.0000x reference)
import jax
import jax.numpy as jnp
from jax.experimental import pallas as pl
from jax.experimental.pallas import tpu as pltpu
from jax.experimental.pallas import tpu_sc as plsc

_B = 16384
_D = 16
_R = 5
_SLICE = _B // 32


def _sc_body(zut_hbm, zit_hbm, pbt_hbm, prm_hbm, puit_hbm, xui_hbm,
             zu_v, zi_v, pbt_v, prm_v, po_v, xo_v, sem):
    c = jax.lax.axis_index("c")
    s = jax.lax.axis_index("s")
    b0 = (c * 16 + s) * _SLICE
    pltpu.async_copy(zut_hbm.at[:, pl.ds(b0, _SLICE)], zu_v, sem).wait()
    pltpu.async_copy(zit_hbm.at[:, pl.ds(b0, _SLICE)], zi_v, sem).wait()
    pltpu.async_copy(pbt_hbm, pbt_v, sem).wait()
    pltpu.async_copy(prm_hbm, prm_v, sem).wait()

    pv = prm_v[...]

    @pl.loop(0, _SLICE // 16)
    def _chunk(ci):
        base = ci * 16
        zs = [zu_v[j, pl.ds(base, 16)] for j in range(_D)]
        t0a = jnp.zeros((16,), jnp.float32)
        t0b = jnp.zeros((16,), jnp.float32)
        t1a = jnp.zeros((16,), jnp.float32)
        t1b = jnp.zeros((16,), jnp.float32)
        for k in range(_D):
            zik = zi_v[k, pl.ds(base, 16)]

            def _u(sb):
                off = (sb * 256 + k * 16) * 16
                ua = zs[0] * pbt_v[pl.ds(off, 16)]
                ub = zs[1] * pbt_v[pl.ds(off + 16, 16)]
                for j in range(2, _D, 2):
                    ua = ua + zs[j] * pbt_v[pl.ds(off + j * 16, 16)]
                    ub = ub + zs[j + 1] * pbt_v[pl.ds(off + (j + 1) * 16, 16)]
                return ua + ub

            if k % 2 == 0:
                t0a = t0a + _u(0) * zik
                t1a = t1a + _u(1) * zik
            else:
                t0b = t0b + _u(0) * zik
                t1b = t1b + _u(1) * zik
        t0 = t0a + t0b
        t1 = t1a + t1b
        ps = [pv[r] * t0 + pv[_R + r] * t1 for r in range(_R)]
        m = ps[0]
        for r in range(1, _R):
            m = jnp.maximum(m, ps[r])
        es = [jnp.exp(p - m) for p in ps]
        den = es[0]
        num = es[0] * pv[2 * _R]
        for r in range(1, _R):
            den = den + es[r]
            num = num + es[r] * pv[2 * _R + r]
        x = num / den
        for r in range(_R):
            po_v[r, pl.ds(base, 16)] = ps[r]
        xo_v[pl.ds(base, 16)] = x

    pltpu.async_copy(po_v, puit_hbm.at[:, pl.ds(b0, _SLICE)], sem).wait()
    pltpu.async_copy(xo_v, xui_hbm.at[pl.ds(b0, _SLICE)], sem).wait()


def kernel(zu, zi, P, A, relations):
    b, d = zu.shape
    r = relations.shape[0]
    zut = zu.T
    zit = zi.T
    pbt = jnp.broadcast_to(
        jnp.transpose(P, (0, 2, 1)).reshape(2 * d * d)[:, None],
        (2 * d * d, 16)).reshape(2 * d * d * 16)
    at = jnp.transpose(A, (1, 2, 0)).reshape(2 * r)
    prm = jnp.concatenate(
        [at, relations, jnp.zeros((1,), jnp.float32)])
    mesh = plsc.VectorSubcoreMesh(core_axis_name="c", subcore_axis_name="s")
    sck = pl.kernel(
        _sc_body,
        out_type=[
            jax.ShapeDtypeStruct((r, b), jnp.float32),
            jax.ShapeDtypeStruct((b,), jnp.float32),
        ],
        mesh=mesh,
        scratch_types=[
            pltpu.VMEM((d, _SLICE), jnp.float32),
            pltpu.VMEM((d, _SLICE), jnp.float32),
            pltpu.VMEM((2 * d * d * 16,), jnp.float32),
            pltpu.VMEM((d,), jnp.float32),
            pltpu.VMEM((r, _SLICE), jnp.float32),
            pltpu.VMEM((_SLICE,), jnp.float32),
            pltpu.SemaphoreType.DMA,
        ],
    )
    puit, xui = sck(zut, zit, pbt, prm)
    return (xui, puit.T)


# speedup vs baseline: 12.3764x; 12.3764x over previous
import jax
import jax.numpy as jnp
from jax.experimental import pallas as pl

_N = 8192

_CONTRACT0 = (((0,), (0,)), ((), ()))


def _body(zut_ref, zit_ref, pr_ref, at_ref, rel_ref, puit_ref, xui_ref):
    zu_b = zut_ref[...]
    zi_b = zit_ref[...]
    pr = pr_ref[...]
    at = at_ref[...]
    rel = rel_ref[...].reshape(1, 5)
    w = jnp.concatenate([rel, jnp.ones((1, 5), jnp.float32)], axis=0)
    u0 = jax.lax.dot_general(pr[:16, :], zu_b, _CONTRACT0,
                             preferred_element_type=jnp.float32)
    u1 = jax.lax.dot_general(pr[16:, :], zu_b, _CONTRACT0,
                             preferred_element_type=jnp.float32)
    t0 = jnp.sum(u0 * zi_b, axis=0, keepdims=True)
    t1 = jnp.sum(u1 * zi_b, axis=0, keepdims=True)
    t = jnp.concatenate([t0, t1], axis=0)
    p = jax.lax.dot_general(at, t, _CONTRACT0,
                            preferred_element_type=jnp.float32)
    m = jnp.max(p, axis=0, keepdims=True)
    e = jnp.exp(p - m)
    nd = jnp.dot(w, e, preferred_element_type=jnp.float32)
    puit_ref[...] = p
    xui_ref[...] = (nd[0:1, :] / nd[1:2, :]).reshape(-1)


def kernel(zu, zi, P, A, relations):
    b, d = zu.shape
    r = relations.shape[0]
    zut = zu.T
    zit = zi.T
    pr = P.reshape(2 * d, d)
    at = jnp.transpose(A, (1, 2, 0)).reshape(2, r)
    grid = b // _N
    puit, xui = pl.pallas_call(
        _body,
        grid=(grid,),
        in_specs=[
            pl.BlockSpec((d, _N), lambda i: (0, i)),
            pl.BlockSpec((d, _N), lambda i: (0, i)),
            pl.BlockSpec((2 * d, d), lambda i: (0, 0)),
            pl.BlockSpec((2, r), lambda i: (0, 0)),
            pl.BlockSpec((r,), lambda i: (0,)),
        ],
        out_specs=[
            pl.BlockSpec((r, _N), lambda i: (0, i)),
            pl.BlockSpec((_N,), lambda i: (i,)),
        ],
        out_shape=[
            jax.ShapeDtypeStruct((r, b), jnp.float32),
            jax.ShapeDtypeStruct((b,), jnp.float32),
        ],
    )(zut, zit, pr, at, relations)
    return (xui, puit.T)


# speedup vs baseline: 13.0392x; 1.0536x over previous
import jax
import jax.numpy as jnp
from jax.experimental import pallas as pl

_N = 16384

_CONTRACT0 = (((0,), (0,)), ((), ()))


def _body(zut_ref, zit_ref, pr_ref, at_ref, rel_ref, puit_ref, xui_ref):
    zu_b = zut_ref[...]
    zi_b = zit_ref[...]
    pr = pr_ref[...]
    at = at_ref[...]
    rel = rel_ref[...].reshape(1, 5)
    w = jnp.concatenate([rel, jnp.ones((1, 5), jnp.float32)], axis=0)
    u0 = jax.lax.dot_general(pr[:16, :], zu_b, _CONTRACT0,
                             preferred_element_type=jnp.float32)
    u1 = jax.lax.dot_general(pr[16:, :], zu_b, _CONTRACT0,
                             preferred_element_type=jnp.float32)
    t0 = jnp.sum(u0 * zi_b, axis=0, keepdims=True)
    t1 = jnp.sum(u1 * zi_b, axis=0, keepdims=True)
    t = jnp.concatenate([t0, t1], axis=0)
    p = jax.lax.dot_general(at, t, _CONTRACT0,
                            preferred_element_type=jnp.float32)
    m = jnp.max(p, axis=0, keepdims=True)
    e = jnp.exp(p - m)
    nd = jnp.dot(w, e, preferred_element_type=jnp.float32)
    puit_ref[...] = p
    xui_ref[...] = (nd[0:1, :] / nd[1:2, :]).reshape(-1)


def kernel(zu, zi, P, A, relations):
    b, d = zu.shape
    r = relations.shape[0]
    zut = zu.T
    zit = zi.T
    pr = P.reshape(2 * d, d)
    at = jnp.transpose(A, (1, 2, 0)).reshape(2, r)
    grid = b // _N
    puit, xui = pl.pallas_call(
        _body,
        grid=(grid,),
        in_specs=[
            pl.BlockSpec((d, _N), lambda i: (0, i)),
            pl.BlockSpec((d, _N), lambda i: (0, i)),
            pl.BlockSpec((2 * d, d), lambda i: (0, 0)),
            pl.BlockSpec((2, r), lambda i: (0, 0)),
            pl.BlockSpec((r,), lambda i: (0,)),
        ],
        out_specs=[
            pl.BlockSpec((r, _N), lambda i: (0, i)),
            pl.BlockSpec((_N,), lambda i: (i,)),
        ],
        out_shape=[
            jax.ShapeDtypeStruct((r, b), jnp.float32),
            jax.ShapeDtypeStruct((b,), jnp.float32),
        ],
    )(zut, zit, pr, at, relations)
    return (xui, puit.T)
